# baseline (speedup 1.0000x reference)
import functools

import jax
import jax.numpy as jnp
from jax import lax
from jax.experimental import pallas as pl
from jax.experimental.pallas import tpu as pltpu
from jax.experimental.pallas import tpu_sc as plsc

_M = 16
_B = 8
_N = 1250
_BN = _B * _N
_D = 3
_E = 160000
_KM_ITERS = 20
_TOPK = 4

_NW = 32
_CHUNK = 128
_NCHUNK = 40
_EPW = _CHUNK * _NCHUNK
_EPAD = _NW * _EPW
_NBUF = 8
_DUMMY_ROW = _BN
_ACC_ROWS = _BN + _M
_STRIPE = 624
_ZTAIL = _ACC_ROWS - 15 * _STRIPE
_CTAIL = _BN - 15 * _STRIPE


def _pool_body(x3t_ref, f_ref, cinit_ref, cent_ref, xp_ref, srow_ref, scol_ref):
    x3a = x3t_ref[0]
    x3 = x3a[:_D]
    cent0 = cinit_ref[0]

    def dist(cent):
        d = None
        for c in range(_D):
            diff = x3[c:c + 1, :] - cent[:, c:c + 1]
            sq = diff * diff
            d = sq if d is None else d + sq
        return d

    miota = lax.broadcasted_iota(jnp.int32, (_M, _N), 0)

    def step(_, cent):
        d = dist(cent)
        dmin = jnp.min(d, axis=0, keepdims=True)
        first = jnp.min(jnp.where(d == dmin, miota, _M), axis=0, keepdims=True)
        p = (miota == first).astype(jnp.float32)
        sums4 = lax.dot_general(p, x3a, (((1,), (1,)), ((), ())),
                                precision=lax.Precision.HIGHEST)
        sums = sums4[:, :_D]
        cnt = sums4[:, _D:]
        mean = sums / jnp.maximum(cnt, 1.0)
        return jnp.where(cnt > 0, mean, cent)

    cent = lax.fori_loop(0, _KM_ITERS, step, cent0)

    s = -dist(cent)
    smax = jnp.max(s, axis=0, keepdims=True)
    e = jnp.exp(s - smax)
    S = e / jnp.sum(e, axis=0, keepdims=True)

    cent_ref[0] = cent
    xp_ref[0] = lax.dot_general(S, f_ref[0], (((1,), (0,)), ((), ())),
                                precision=lax.Precision.HIGHEST)
    scol_ref[0] = S
    srow_ref[0] = S.T


def _pool_call(x3t, f, cinit):
    nf = f.shape[2]
    return pl.pallas_call(
        _pool_body,
        grid=(_B,),
        in_specs=[
            pl.BlockSpec((1, _D + 1, _N), lambda b: (b, 0, 0)),
            pl.BlockSpec((1, _N, nf), lambda b: (b, 0, 0)),
            pl.BlockSpec((1, _M, _D), lambda b: (b, 0, 0)),
        ],
        out_specs=[
            pl.BlockSpec((1, _M, _D), lambda b: (b, 0, 0)),
            pl.BlockSpec((1, _M, nf), lambda b: (b, 0, 0)),
            pl.BlockSpec((1, _N, _M), lambda b: (b, 0, 0)),
            pl.BlockSpec((1, _M, _N), lambda b: (b, 0, 0)),
        ],
        out_shape=[
            jax.ShapeDtypeStruct((_B, _M, _D), jnp.float32),
            jax.ShapeDtypeStruct((_B, _M, nf), jnp.float32),
            jax.ShapeDtypeStruct((_B, _N, _M), jnp.float32),
            jax.ShapeDtypeStruct((_B, _M, _N), jnp.float32),
        ],
    )(x3t, f, cinit)


def _scatter_call(s_rows, e0p, e1p, zrows):
    mesh = plsc.VectorSubcoreMesh(core_axis_name="c", subcore_axis_name="s")

    @functools.partial(
        pl.kernel,
        out_type=jax.ShapeDtypeStruct((2 * _BN, _M), jnp.float32),
        mesh=mesh,
        scratch_types=[
            pltpu.VMEM((_NCHUNK, _CHUNK), jnp.int32),
            pltpu.VMEM((_NCHUNK, _CHUNK), jnp.int32),
            pltpu.VMEM((_NBUF, _CHUNK, _M), jnp.float32),
            pltpu.VMEM_SHARED((_ACC_ROWS, _M), jnp.float32),
            pltpu.SemaphoreType.DMA,
        ],
        compiler_params=pltpu.CompilerParams(use_tc_tiling_on_sc=False),
    )
    def scatter_kernel(s_hbm, e0_hbm, e1_hbm, z_hbm, out_hbm,
                       idx0_v, idx1_v, rows_v, acc_sh, sem):
        cid = lax.axis_index("c")
        sid = lax.axis_index("s")
        wid = sid * 2 + cid

        @pl.when(sid < 15)
        def _():
            pltpu.sync_copy(z_hbm.at[pl.ds(0, _STRIPE)],
                            acc_sh.at[pl.ds(sid * _STRIPE, _STRIPE)])

        @pl.when(sid == 15)
        def _():
            pltpu.sync_copy(z_hbm, acc_sh.at[pl.ds(15 * _STRIPE, _ZTAIL)])

        plsc.subcore_barrier()

        pltpu.sync_copy(e0_hbm.at[wid], idx0_v)
        pltpu.sync_copy(e1_hbm.at[wid], idx1_v)

        def gather(j, b):
            return pltpu.make_async_copy(
                s_hbm.at[idx1_v.at[j]], rows_v.at[b], sem)

        for b in range(_NBUF):
            gather(b, b).start()

        def group(g, carry):
            for b in range(_NBUF):
                j = g * _NBUF + b
                gather(j, b).wait()
                pltpu.sync_copy(rows_v.at[b], acc_sh.at[idx0_v.at[j]],
                                add=True)

                @pl.when(j + _NBUF < _NCHUNK)
                def _():
                    gather(j + _NBUF, b).start()
            return carry

        lax.fori_loop(0, _NCHUNK // _NBUF, group, 0)
        plsc.subcore_barrier()

        @pl.when(sid < 15)
        def _():
            pltpu.sync_copy(acc_sh.at[pl.ds(sid * _STRIPE, _STRIPE)],
                            out_hbm.at[pl.ds(cid * _BN + sid * _STRIPE, _STRIPE)])

        @pl.when(sid == 15)
        def _():
            pltpu.sync_copy(acc_sh.at[pl.ds(15 * _STRIPE, _CTAIL)],
                            out_hbm.at[pl.ds(cid * _BN + 15 * _STRIPE, _CTAIL)])

    return scatter_kernel(s_rows, e0p, e1p, zrows)


def _amm_body(scol_ref, as0_ref, as1_ref, dst_ref):
    S = scol_ref[0]
    asb = as0_ref[0] + as1_ref[0]
    a = lax.dot_general(S, asb, (((1,), (0,)), ((), ())),
                        precision=lax.Precision.HIGHEST)
    liota = lax.broadcasted_iota(jnp.int32, (_M, _M), 1)
    cols = []
    for _ in range(_TOPK):
        vmax = jnp.max(a, axis=1, keepdims=True)
        first = jnp.min(jnp.where(a == vmax, liota, _M), axis=1, keepdims=True)
        cols.append(first)
        a = jnp.where(liota == first, -jnp.inf, a)
    dst_ref[0] = jnp.concatenate(cols, axis=1)


def _amm_call(s_cols, as0, as1):
    return pl.pallas_call(
        _amm_body,
        grid=(_B,),
        in_specs=[
            pl.BlockSpec((1, _M, _N), lambda b: (b, 0, 0)),
            pl.BlockSpec((1, _N, _M), lambda b: (b, 0, 0)),
            pl.BlockSpec((1, _N, _M), lambda b: (b, 0, 0)),
        ],
        out_specs=pl.BlockSpec((1, _M, _TOPK), lambda b: (b, 0, 0)),
        out_shape=jax.ShapeDtypeStruct((_B, _M, _TOPK), jnp.int32),
    )(s_cols, as0, as1)


def kernel(x, e_, b_):
    nc = x.shape[1]
    x3b = x[:, :_D].reshape(_B, _N, _D)
    x3t = jnp.concatenate(
        [x3b.transpose(0, 2, 1), jnp.ones((_B, 1, _N), jnp.float32)], axis=1)
    cinit = x3b[:, :_M, :]
    f = x[:, _D:].reshape(_B, _N, nc - _D)

    cent, xp, s_rows, s_cols = _pool_call(x3t, f, cinit)

    pad = _EPAD - _E
    e0p = jnp.concatenate(
        [e_[0], jnp.full((pad,), _DUMMY_ROW, jnp.int32)]
    ).reshape(_NW, _NCHUNK, _CHUNK)
    e1p = jnp.concatenate(
        [e_[1], jnp.zeros((pad,), jnp.int32)]
    ).reshape(_NW, _NCHUNK, _CHUNK)
    zrows = jnp.zeros((_ZTAIL, _M), jnp.float32)
    as2 = _scatter_call(s_rows.reshape(_BN, _M), e0p, e1p, zrows)

    dst = _amm_call(s_cols, as2[:_BN].reshape(_B, _N, _M),
                    as2[_BN:].reshape(_B, _N, _M))

    x_out = jnp.concatenate(
        [cent.reshape(_B * _M, _D), xp.reshape(_B * _M, nc - _D)], axis=1)
    off = (jnp.arange(_B, dtype=jnp.int32) * _M).reshape(_B, 1, 1)
    src = jnp.broadcast_to(
        jnp.arange(_M, dtype=jnp.int32)[None, :, None], (_B, _M, _TOPK)) + off
    e_out = jnp.stack([src.reshape(-1), (dst + off).reshape(-1)], axis=0)
    b_out = jnp.repeat(jnp.arange(_B, dtype=jnp.int32), _M)
    return (x_out, e_out, b_out)


# speedup vs baseline: 1.0005x; 1.0005x over previous
import functools

import jax
import jax.numpy as jnp
from jax import lax
from jax.experimental import pallas as pl
from jax.experimental.pallas import tpu as pltpu
from jax.experimental.pallas import tpu_sc as plsc

_M = 16
_B = 8
_N = 1250
_BN = _B * _N
_D = 3
_E = 160000
_KM_ITERS = 20
_TOPK = 4

_NW = 32
_CHUNK = 128
_NCHUNK = 40
_EPW = _CHUNK * _NCHUNK
_EPAD = _NW * _EPW
_NBUF = 4
_DUMMY_ROW = _BN
_ACC_ROWS = _BN + _M
_STRIPE = 624
_ZTAIL = _ACC_ROWS - 15 * _STRIPE
_CTAIL = _BN - 15 * _STRIPE


def _pool_body(x3t_ref, f_ref, cinit_ref, cent_ref, xp_ref, srow_ref, scol_ref):
    x3a = x3t_ref[0]
    x3 = x3a[:_D]
    cent0 = cinit_ref[0]

    def dist(cent):
        d = None
        for c in range(_D):
            diff = x3[c:c + 1, :] - cent[:, c:c + 1]
            sq = diff * diff
            d = sq if d is None else d + sq
        return d

    miota = lax.broadcasted_iota(jnp.int32, (_M, _N), 0)

    def step(_, cent):
        d = dist(cent)
        dmin = jnp.min(d, axis=0, keepdims=True)
        first = jnp.min(jnp.where(d == dmin, miota, _M), axis=0, keepdims=True)
        p = (miota == first).astype(jnp.float32)
        sums4 = lax.dot_general(p, x3a, (((1,), (1,)), ((), ())),
                                precision=lax.Precision.HIGHEST)
        sums = sums4[:, :_D]
        cnt = sums4[:, _D:]
        mean = sums / jnp.maximum(cnt, 1.0)
        return jnp.where(cnt > 0, mean, cent)

    cent = lax.fori_loop(0, _KM_ITERS, step, cent0)

    s = -dist(cent)
    smax = jnp.max(s, axis=0, keepdims=True)
    e = jnp.exp(s - smax)
    S = e / jnp.sum(e, axis=0, keepdims=True)

    cent_ref[0] = cent
    xp_ref[0] = lax.dot_general(S, f_ref[0], (((1,), (0,)), ((), ())),
                                precision=lax.Precision.HIGHEST)
    scol_ref[0] = S
    srow_ref[0] = S.T


def _pool_call(x3t, f, cinit):
    nf = f.shape[2]
    return pl.pallas_call(
        _pool_body,
        grid=(_B,),
        in_specs=[
            pl.BlockSpec((1, _D + 1, _N), lambda b: (b, 0, 0)),
            pl.BlockSpec((1, _N, nf), lambda b: (b, 0, 0)),
            pl.BlockSpec((1, _M, _D), lambda b: (b, 0, 0)),
        ],
        out_specs=[
            pl.BlockSpec((1, _M, _D), lambda b: (b, 0, 0)),
            pl.BlockSpec((1, _M, nf), lambda b: (b, 0, 0)),
            pl.BlockSpec((1, _N, _M), lambda b: (b, 0, 0)),
            pl.BlockSpec((1, _M, _N), lambda b: (b, 0, 0)),
        ],
        out_shape=[
            jax.ShapeDtypeStruct((_B, _M, _D), jnp.float32),
            jax.ShapeDtypeStruct((_B, _M, nf), jnp.float32),
            jax.ShapeDtypeStruct((_B, _N, _M), jnp.float32),
            jax.ShapeDtypeStruct((_B, _M, _N), jnp.float32),
        ],
    )(x3t, f, cinit)


def _scatter_call(s_rows, e0p, e1p, zrows):
    mesh = plsc.VectorSubcoreMesh(core_axis_name="c", subcore_axis_name="s")

    @functools.partial(
        pl.kernel,
        out_type=jax.ShapeDtypeStruct((2 * _BN, _M), jnp.float32),
        mesh=mesh,
        scratch_types=[
            pltpu.VMEM((_NCHUNK, _CHUNK), jnp.int32),
            pltpu.VMEM((_NCHUNK, _CHUNK), jnp.int32),
            pltpu.VMEM((_NBUF, _CHUNK, _M), jnp.float32),
            pltpu.VMEM_SHARED((_ACC_ROWS, _M), jnp.float32),
            pltpu.SemaphoreType.DMA,
        ],
        compiler_params=pltpu.CompilerParams(use_tc_tiling_on_sc=False),
    )
    def scatter_kernel(s_hbm, e0_hbm, e1_hbm, z_hbm, out_hbm,
                       idx0_v, idx1_v, rows_v, acc_sh, sem):
        cid = lax.axis_index("c")
        sid = lax.axis_index("s")
        wid = sid * 2 + cid

        @pl.when(sid < 15)
        def _():
            pltpu.sync_copy(z_hbm.at[pl.ds(0, _STRIPE)],
                            acc_sh.at[pl.ds(sid * _STRIPE, _STRIPE)])

        @pl.when(sid == 15)
        def _():
            pltpu.sync_copy(z_hbm, acc_sh.at[pl.ds(15 * _STRIPE, _ZTAIL)])

        plsc.subcore_barrier()

        pltpu.sync_copy(e0_hbm.at[wid], idx0_v)
        pltpu.sync_copy(e1_hbm.at[wid], idx1_v)

        def gather(j, b):
            return pltpu.make_async_copy(
                s_hbm.at[idx1_v.at[j]], rows_v.at[b], sem)

        for b in range(_NBUF):
            gather(b, b).start()

        def group(g, carry):
            for b in range(_NBUF):
                j = g * _NBUF + b
                gather(j, b).wait()
                pltpu.sync_copy(rows_v.at[b], acc_sh.at[idx0_v.at[j]],
                                add=True)

                @pl.when(j + _NBUF < _NCHUNK)
                def _():
                    gather(j + _NBUF, b).start()
            return carry

        lax.fori_loop(0, _NCHUNK // _NBUF, group, 0)
        plsc.subcore_barrier()

        @pl.when(sid < 15)
        def _():
            pltpu.sync_copy(acc_sh.at[pl.ds(sid * _STRIPE, _STRIPE)],
                            out_hbm.at[pl.ds(cid * _BN + sid * _STRIPE, _STRIPE)])

        @pl.when(sid == 15)
        def _():
            pltpu.sync_copy(acc_sh.at[pl.ds(15 * _STRIPE, _CTAIL)],
                            out_hbm.at[pl.ds(cid * _BN + 15 * _STRIPE, _CTAIL)])

    return scatter_kernel(s_rows, e0p, e1p, zrows)


def _amm_body(scol_ref, as0_ref, as1_ref, dst_ref):
    S = scol_ref[0]
    asb = as0_ref[0] + as1_ref[0]
    a = lax.dot_general(S, asb, (((1,), (0,)), ((), ())),
                        precision=lax.Precision.HIGHEST)
    liota = lax.broadcasted_iota(jnp.int32, (_M, _M), 1)
    cols = []
    for _ in range(_TOPK):
        vmax = jnp.max(a, axis=1, keepdims=True)
        first = jnp.min(jnp.where(a == vmax, liota, _M), axis=1, keepdims=True)
        cols.append(first)
        a = jnp.where(liota == first, -jnp.inf, a)
    dst_ref[0] = jnp.concatenate(cols, axis=1)


def _amm_call(s_cols, as0, as1):
    return pl.pallas_call(
        _amm_body,
        grid=(_B,),
        in_specs=[
            pl.BlockSpec((1, _M, _N), lambda b: (b, 0, 0)),
            pl.BlockSpec((1, _N, _M), lambda b: (b, 0, 0)),
            pl.BlockSpec((1, _N, _M), lambda b: (b, 0, 0)),
        ],
        out_specs=pl.BlockSpec((1, _M, _TOPK), lambda b: (b, 0, 0)),
        out_shape=jax.ShapeDtypeStruct((_B, _M, _TOPK), jnp.int32),
    )(s_cols, as0, as1)


def kernel(x, e_, b_):
    nc = x.shape[1]
    x3b = x[:, :_D].reshape(_B, _N, _D)
    x3t = jnp.concatenate(
        [x3b.transpose(0, 2, 1), jnp.ones((_B, 1, _N), jnp.float32)], axis=1)
    cinit = x3b[:, :_M, :]
    f = x[:, _D:].reshape(_B, _N, nc - _D)

    cent, xp, s_rows, s_cols = _pool_call(x3t, f, cinit)

    pad = _EPAD - _E
    e0p = jnp.concatenate(
        [e_[0], jnp.full((pad,), _DUMMY_ROW, jnp.int32)]
    ).reshape(_NW, _NCHUNK, _CHUNK)
    e1p = jnp.concatenate(
        [e_[1], jnp.zeros((pad,), jnp.int32)]
    ).reshape(_NW, _NCHUNK, _CHUNK)
    zrows = jnp.zeros((_ZTAIL, _M), jnp.float32)
    as2 = _scatter_call(s_rows.reshape(_BN, _M), e0p, e1p, zrows)

    dst = _amm_call(s_cols, as2[:_BN].reshape(_B, _N, _M),
                    as2[_BN:].reshape(_B, _N, _M))

    x_out = jnp.concatenate(
        [cent.reshape(_B * _M, _D), xp.reshape(_B * _M, nc - _D)], axis=1)
    off = (jnp.arange(_B, dtype=jnp.int32) * _M).reshape(_B, 1, 1)
    src = jnp.broadcast_to(
        jnp.arange(_M, dtype=jnp.int32)[None, :, None], (_B, _M, _TOPK)) + off
    e_out = jnp.stack([src.reshape(-1), (dst + off).reshape(-1)], axis=0)
    b_out = jnp.repeat(jnp.arange(_B, dtype=jnp.int32), _M)
    return (x_out, e_out, b_out)


# speedup vs baseline: 1.0620x; 1.0615x over previous
import functools

import jax
import jax.numpy as jnp
from jax import lax
from jax.experimental import pallas as pl
from jax.experimental.pallas import tpu as pltpu
from jax.experimental.pallas import tpu_sc as plsc

_M = 16
_B = 8
_N = 1250
_BN = _B * _N
_D = 3
_E = 160000
_KM_ITERS = 20
_TOPK = 4

_NW = 32
_CHUNK = 128
_NCHUNK = 40
_EPW = _CHUNK * _NCHUNK
_EPAD = _NW * _EPW
_NBUF = 4
_DUMMY_ROW = _BN
_ACC_ROWS = _BN + _M
_STRIPE = 624
_ZTAIL = _ACC_ROWS - 15 * _STRIPE
_CTAIL = _BN - 15 * _STRIPE


def _pool_body(x3t_ref, f_ref, cinit_ref, cent_ref, xp_ref, srow_ref, scol_ref):
    x3 = x3t_ref[0]
    cent0 = cinit_ref[0]

    def dist(cent):
        d = None
        for c in range(_D):
            diff = x3[c:c + 1, :] - cent[:, c:c + 1]
            sq = diff * diff
            d = sq if d is None else d + sq
        return d

    miota = lax.broadcasted_iota(jnp.int32, (_M, _N), 0)

    def step(_, cent):
        d = dist(cent)
        dmin = jnp.min(d, axis=0, keepdims=True)
        first = jnp.min(jnp.where(d == dmin, miota, _M), axis=0, keepdims=True)
        p = (miota == first).astype(jnp.float32)
        sums = lax.dot_general(p, x3, (((1,), (1,)), ((), ())),
                               precision=lax.Precision.HIGHEST)
        cnt = jnp.sum(p, axis=1, keepdims=True)
        mean = sums / jnp.maximum(cnt, 1.0)
        return jnp.where(cnt > 0, mean, cent)

    cent = lax.fori_loop(0, _KM_ITERS, step, cent0)

    s = -dist(cent)
    smax = jnp.max(s, axis=0, keepdims=True)
    e = jnp.exp(s - smax)
    S = e / jnp.sum(e, axis=0, keepdims=True)

    cent_ref[0] = cent
    xp_ref[0] = lax.dot_general(S, f_ref[0], (((1,), (0,)), ((), ())),
                                precision=lax.Precision.HIGHEST)
    scol_ref[0] = S
    srow_ref[0] = S.T


def _pool_call(x3t, f, cinit):
    nf = f.shape[2]
    return pl.pallas_call(
        _pool_body,
        grid=(_B,),
        in_specs=[
            pl.BlockSpec((1, _D, _N), lambda b: (b, 0, 0)),
            pl.BlockSpec((1, _N, nf), lambda b: (b, 0, 0)),
            pl.BlockSpec((1, _M, _D), lambda b: (b, 0, 0)),
        ],
        out_specs=[
            pl.BlockSpec((1, _M, _D), lambda b: (b, 0, 0)),
            pl.BlockSpec((1, _M, nf), lambda b: (b, 0, 0)),
            pl.BlockSpec((1, _N, _M), lambda b: (b, 0, 0)),
            pl.BlockSpec((1, _M, _N), lambda b: (b, 0, 0)),
        ],
        out_shape=[
            jax.ShapeDtypeStruct((_B, _M, _D), jnp.float32),
            jax.ShapeDtypeStruct((_B, _M, nf), jnp.float32),
            jax.ShapeDtypeStruct((_B, _N, _M), jnp.float32),
            jax.ShapeDtypeStruct((_B, _M, _N), jnp.float32),
        ],
    )(x3t, f, cinit)


def _scatter_call(s_rows, e0p, e1p, zrows):
    mesh = plsc.VectorSubcoreMesh(core_axis_name="c", subcore_axis_name="s")

    @functools.partial(
        pl.kernel,
        out_type=jax.ShapeDtypeStruct((2 * _BN, _M), jnp.float32),
        mesh=mesh,
        scratch_types=[
            pltpu.VMEM((_NCHUNK, _CHUNK), jnp.int32),
            pltpu.VMEM((_NCHUNK, _CHUNK), jnp.int32),
            pltpu.VMEM((_NBUF, _CHUNK, _M), jnp.float32),
            pltpu.VMEM_SHARED((_ACC_ROWS, _M), jnp.float32),
            pltpu.SemaphoreType.DMA,
        ],
        compiler_params=pltpu.CompilerParams(use_tc_tiling_on_sc=False),
    )
    def scatter_kernel(s_hbm, e0_hbm, e1_hbm, z_hbm, out_hbm,
                       idx0_v, idx1_v, rows_v, acc_sh, sem):
        cid = lax.axis_index("c")
        sid = lax.axis_index("s")
        wid = sid * 2 + cid

        @pl.when(sid < 15)
        def _():
            pltpu.sync_copy(z_hbm.at[pl.ds(0, _STRIPE)],
                            acc_sh.at[pl.ds(sid * _STRIPE, _STRIPE)])

        @pl.when(sid == 15)
        def _():
            pltpu.sync_copy(z_hbm, acc_sh.at[pl.ds(15 * _STRIPE, _ZTAIL)])

        plsc.subcore_barrier()

        pltpu.sync_copy(e0_hbm.at[wid], idx0_v)
        pltpu.sync_copy(e1_hbm.at[wid], idx1_v)

        def gather(j, b):
            return pltpu.make_async_copy(
                s_hbm.at[idx1_v.at[j]], rows_v.at[b], sem)

        for b in range(_NBUF):
            gather(b, b).start()

        def group(g, carry):
            for b in range(_NBUF):
                j = g * _NBUF + b
                gather(j, b).wait()
                pltpu.sync_copy(rows_v.at[b], acc_sh.at[idx0_v.at[j]],
                                add=True)

                @pl.when(j + _NBUF < _NCHUNK)
                def _():
                    gather(j + _NBUF, b).start()
            return carry

        lax.fori_loop(0, _NCHUNK // _NBUF, group, 0)
        plsc.subcore_barrier()

        @pl.when(sid < 15)
        def _():
            pltpu.sync_copy(acc_sh.at[pl.ds(sid * _STRIPE, _STRIPE)],
                            out_hbm.at[pl.ds(cid * _BN + sid * _STRIPE, _STRIPE)])

        @pl.when(sid == 15)
        def _():
            pltpu.sync_copy(acc_sh.at[pl.ds(15 * _STRIPE, _CTAIL)],
                            out_hbm.at[pl.ds(cid * _BN + 15 * _STRIPE, _CTAIL)])

    return scatter_kernel(s_rows, e0p, e1p, zrows)


def _amm_body(scol_ref, as0_ref, as1_ref, dst_ref):
    S = scol_ref[0]
    asb = as0_ref[0] + as1_ref[0]
    a = lax.dot_general(S, asb, (((1,), (0,)), ((), ())),
                        precision=lax.Precision.HIGHEST)
    liota = lax.broadcasted_iota(jnp.int32, (_M, _M), 1)
    cols = []
    for _ in range(_TOPK):
        vmax = jnp.max(a, axis=1, keepdims=True)
        first = jnp.min(jnp.where(a == vmax, liota, _M), axis=1, keepdims=True)
        cols.append(first)
        a = jnp.where(liota == first, -jnp.inf, a)
    dst_ref[0] = jnp.concatenate(cols, axis=1)


def _amm_call(s_cols, as0, as1):
    return pl.pallas_call(
        _amm_body,
        grid=(_B,),
        in_specs=[
            pl.BlockSpec((1, _M, _N), lambda b: (b, 0, 0)),
            pl.BlockSpec((1, _N, _M), lambda b: (b, 0, 0)),
            pl.BlockSpec((1, _N, _M), lambda b: (b, 0, 0)),
        ],
        out_specs=pl.BlockSpec((1, _M, _TOPK), lambda b: (b, 0, 0)),
        out_shape=jax.ShapeDtypeStruct((_B, _M, _TOPK), jnp.int32),
    )(s_cols, as0, as1)


def kernel(x, e_, b_):
    nc = x.shape[1]
    x3b = x[:, :_D].reshape(_B, _N, _D)
    x3t = x3b.transpose(0, 2, 1)
    cinit = x3b[:, :_M, :]
    f = x[:, _D:].reshape(_B, _N, nc - _D)

    cent, xp, s_rows, s_cols = _pool_call(x3t, f, cinit)

    pad = _EPAD - _E
    e0p = jnp.concatenate(
        [e_[0], jnp.full((pad,), _DUMMY_ROW, jnp.int32)]
    ).reshape(_NW, _NCHUNK, _CHUNK)
    e1p = jnp.concatenate(
        [e_[1], jnp.zeros((pad,), jnp.int32)]
    ).reshape(_NW, _NCHUNK, _CHUNK)
    zrows = jnp.zeros((_ZTAIL, _M), jnp.float32)
    as2 = _scatter_call(s_rows.reshape(_BN, _M), e0p, e1p, zrows)

    dst = _amm_call(s_cols, as2[:_BN].reshape(_B, _N, _M),
                    as2[_BN:].reshape(_B, _N, _M))

    x_out = jnp.concatenate(
        [cent.reshape(_B * _M, _D), xp.reshape(_B * _M, nc - _D)], axis=1)
    off = (jnp.arange(_B, dtype=jnp.int32) * _M).reshape(_B, 1, 1)
    src = jnp.broadcast_to(
        jnp.arange(_M, dtype=jnp.int32)[None, :, None], (_B, _M, _TOPK)) + off
    e_out = jnp.stack([src.reshape(-1), (dst + off).reshape(-1)], axis=0)
    b_out = jnp.repeat(jnp.arange(_B, dtype=jnp.int32), _M)
    return (x_out, e_out, b_out)


# speedup vs baseline: 1.3681x; 1.2881x over previous
import functools

import jax
import jax.numpy as jnp
from jax import lax
from jax.experimental import pallas as pl
from jax.experimental.pallas import tpu as pltpu
from jax.experimental.pallas import tpu_sc as plsc

_M = 16
_B = 8
_G = 2
_N = 1250
_BN = _B * _N
_D = 3
_E = 160000
_KM_ITERS = 20
_TOPK = 4

_NW = 32
_CHUNK = 128
_NCHUNK = 40
_EPW = _CHUNK * _NCHUNK
_EPAD = _NW * _EPW
_NBUF = 4
_DUMMY_ROW = _BN
_ACC_ROWS = _BN + _M
_STRIPE = 624
_ZTAIL = _ACC_ROWS - 15 * _STRIPE
_CTAIL = _BN - 15 * _STRIPE


def _pool_body(x_ref, xout_ref, srow_ref, scol_ref):
    xb = x_ref[0]
    x3t = jnp.transpose(xb[:, :, :_D], (0, 2, 1))
    cent0 = xb[:, :_M, :_D]

    def dist(cent):
        d = None
        for c in range(_D):
            diff = x3t[:, c:c + 1, :] - cent[:, :, c:c + 1]
            sq = diff * diff
            d = sq if d is None else d + sq
        return d

    miota = lax.broadcasted_iota(jnp.int32, (_G, _M, _N), 1)

    def step(_, cent):
        d = dist(cent)
        dmin = jnp.min(d, axis=1, keepdims=True)
        first = jnp.min(jnp.where(d == dmin, miota, _M), axis=1, keepdims=True)
        p = (miota == first).astype(jnp.float32)
        sums = lax.dot_general(p, x3t, (((2,), (2,)), ((0,), (0,))),
                               precision=lax.Precision.HIGHEST)
        cnt = jnp.sum(p, axis=2, keepdims=True)
        mean = sums / jnp.maximum(cnt, 1.0)
        return jnp.where(cnt > 0, mean, cent)

    cent = lax.fori_loop(0, _KM_ITERS, step, cent0)

    s = -dist(cent)
    smax = jnp.max(s, axis=1, keepdims=True)
    e = jnp.exp(s - smax)
    S = e / jnp.sum(e, axis=1, keepdims=True)

    xp = lax.dot_general(S, xb[:, :, _D:], (((2,), (1,)), ((0,), (0,))),
                         precision=lax.Precision.HIGHEST)
    xout_ref[0] = jnp.concatenate([cent, xp], axis=2)
    scol_ref[0] = S
    srow_ref[0] = jnp.transpose(S, (0, 2, 1))


def _pool_call(xg):
    ng, nc = xg.shape[0], xg.shape[3]
    return pl.pallas_call(
        _pool_body,
        grid=(ng,),
        in_specs=[
            pl.BlockSpec((1, _G, _N, nc), lambda b: (b, 0, 0, 0)),
        ],
        out_specs=[
            pl.BlockSpec((1, _G, _M, nc), lambda b: (b, 0, 0, 0)),
            pl.BlockSpec((1, _G, _N, _M), lambda b: (b, 0, 0, 0)),
            pl.BlockSpec((1, _G, _M, _N), lambda b: (b, 0, 0, 0)),
        ],
        out_shape=[
            jax.ShapeDtypeStruct((ng, _G, _M, nc), jnp.float32),
            jax.ShapeDtypeStruct((ng, _G, _N, _M), jnp.float32),
            jax.ShapeDtypeStruct((ng, _G, _M, _N), jnp.float32),
        ],
    )(xg)


def _scatter_call(s_rows, ep, zrows):
    mesh = plsc.VectorSubcoreMesh(core_axis_name="c", subcore_axis_name="s")

    @functools.partial(
        pl.kernel,
        out_type=jax.ShapeDtypeStruct((2 * _BN, _M), jnp.float32),
        mesh=mesh,
        scratch_types=[
            pltpu.VMEM((_NCHUNK, _CHUNK), jnp.int32),
            pltpu.VMEM((_NCHUNK, _CHUNK), jnp.int32),
            pltpu.VMEM((_NBUF, _CHUNK, _M), jnp.float32),
            pltpu.VMEM_SHARED((_ACC_ROWS, _M), jnp.float32),
            pltpu.SemaphoreType.DMA,
        ],
        compiler_params=pltpu.CompilerParams(use_tc_tiling_on_sc=False),
    )
    def scatter_kernel(s_hbm, e_hbm, z_hbm, out_hbm,
                       idx0_v, idx1_v, rows_v, acc_sh, sem):
        cid = lax.axis_index("c")
        sid = lax.axis_index("s")
        wid = sid * 2 + cid

        @pl.when(sid < 15)
        def _():
            pltpu.sync_copy(z_hbm.at[pl.ds(0, _STRIPE)],
                            acc_sh.at[pl.ds(sid * _STRIPE, _STRIPE)])

        @pl.when(sid == 15)
        def _():
            pltpu.sync_copy(z_hbm, acc_sh.at[pl.ds(15 * _STRIPE, _ZTAIL)])

        plsc.subcore_barrier()

        pltpu.sync_copy(e_hbm.at[0, wid], idx0_v)
        pltpu.sync_copy(e_hbm.at[1, wid], idx1_v)

        def gather(j, b):
            return pltpu.make_async_copy(
                s_hbm.at[idx1_v.at[j]], rows_v.at[b], sem)

        for b in range(_NBUF):
            gather(b, b).start()

        def group(g, carry):
            for b in range(_NBUF):
                j = g * _NBUF + b
                gather(j, b).wait()
                pltpu.sync_copy(rows_v.at[b], acc_sh.at[idx0_v.at[j]],
                                add=True)

                @pl.when(j + _NBUF < _NCHUNK)
                def _():
                    gather(j + _NBUF, b).start()
            return carry

        lax.fori_loop(0, _NCHUNK // _NBUF, group, 0)
        plsc.subcore_barrier()

        @pl.when(sid < 15)
        def _():
            pltpu.sync_copy(acc_sh.at[pl.ds(sid * _STRIPE, _STRIPE)],
                            out_hbm.at[pl.ds(cid * _BN + sid * _STRIPE, _STRIPE)])

        @pl.when(sid == 15)
        def _():
            pltpu.sync_copy(acc_sh.at[pl.ds(15 * _STRIPE, _CTAIL)],
                            out_hbm.at[pl.ds(cid * _BN + 15 * _STRIPE, _CTAIL)])

    return scatter_kernel(s_rows, ep, zrows)


def _amm_body(scol_ref, as4_ref, dst_ref):
    S = scol_ref[0]
    asb = as4_ref[0, 0] + as4_ref[1, 0]
    a = lax.dot_general(S, asb, (((1,), (0,)), ((), ())),
                        precision=lax.Precision.HIGHEST)
    liota = lax.broadcasted_iota(jnp.int32, (_M, _M), 1)
    cols = []
    for _ in range(_TOPK):
        vmax = jnp.max(a, axis=1, keepdims=True)
        first = jnp.min(jnp.where(a == vmax, liota, _M), axis=1, keepdims=True)
        cols.append(first)
        a = jnp.where(liota == first, -jnp.inf, a)
    dst_ref[0] = jnp.concatenate(cols, axis=1)


def _amm_call(s_cols, as4):
    return pl.pallas_call(
        _amm_body,
        grid=(_B,),
        in_specs=[
            pl.BlockSpec((1, _M, _N), lambda b: (b, 0, 0)),
            pl.BlockSpec((2, 1, _N, _M), lambda b: (0, b, 0, 0)),
        ],
        out_specs=pl.BlockSpec((1, _M, _TOPK), lambda b: (b, 0, 0)),
        out_shape=jax.ShapeDtypeStruct((_B, _M, _TOPK), jnp.int32),
    )(s_cols, as4)


def kernel(x, e_, b_):
    nc = x.shape[1]
    xg = x.reshape(_B // _G, _G, _N, nc)

    xout, s_rows, s_cols = _pool_call(xg)

    pad = _EPAD - _E
    padcols = jnp.concatenate(
        [jnp.full((1, pad), _DUMMY_ROW, jnp.int32),
         jnp.zeros((1, pad), jnp.int32)], axis=0)
    ep = jnp.concatenate([e_, padcols], axis=1).reshape(2, _NW, _NCHUNK, _CHUNK)
    zrows = jnp.zeros((_ZTAIL, _M), jnp.float32)
    as2 = _scatter_call(s_rows.reshape(_BN, _M), ep, zrows)

    dst = _amm_call(s_cols.reshape(_B, _M, _N), as2.reshape(2, _B, _N, _M))

    x_out = xout.reshape(_B * _M, nc)
    off = (jnp.arange(_B, dtype=jnp.int32) * _M).reshape(_B, 1, 1)
    src = jnp.broadcast_to(
        jnp.arange(_M, dtype=jnp.int32)[None, :, None], (_B, _M, _TOPK)) + off
    e_out = jnp.stack([src.reshape(-1), (dst + off).reshape(-1)], axis=0)
    b_out = jnp.repeat(jnp.arange(_B, dtype=jnp.int32), _M)
    return (x_out, e_out, b_out)


# speedup vs baseline: 1.4883x; 1.0879x over previous
import functools

import jax
import jax.numpy as jnp
from jax import lax
from jax.experimental import pallas as pl
from jax.experimental.pallas import tpu as pltpu
from jax.experimental.pallas import tpu_sc as plsc

_M = 16
_B = 8
_G = 8
_N = 1250
_BN = _B * _N
_D = 3
_E = 160000
_KM_ITERS = 20
_TOPK = 4

_NW = 32
_CHUNK = 128
_NCHUNK = 40
_EPW = _CHUNK * _NCHUNK
_EPAD = _NW * _EPW
_NBUF = 4
_DUMMY_ROW = _BN
_ACC_ROWS = _BN + _M
_STRIPE = 624
_ZTAIL = _ACC_ROWS - 15 * _STRIPE
_CTAIL = _BN - 15 * _STRIPE


def _pool_body(x_ref, xout_ref, srow_ref, scol_ref):
    xb = x_ref[0]
    x3t = jnp.transpose(xb[:, :, :_D], (0, 2, 1))
    cent0 = xb[:, :_M, :_D]

    def dist(cent):
        d = None
        for c in range(_D):
            diff = x3t[:, c:c + 1, :] - cent[:, :, c:c + 1]
            sq = diff * diff
            d = sq if d is None else d + sq
        return d

    miota = lax.broadcasted_iota(jnp.int32, (_G, _M, _N), 1)

    def step(_, cent):
        d = dist(cent)
        dmin = jnp.min(d, axis=1, keepdims=True)
        first = jnp.min(jnp.where(d == dmin, miota, _M), axis=1, keepdims=True)
        p = (miota == first).astype(jnp.float32)
        sums = lax.dot_general(p, x3t, (((2,), (2,)), ((0,), (0,))),
                               precision=lax.Precision.HIGHEST)
        cnt = jnp.sum(p, axis=2, keepdims=True)
        mean = sums / jnp.maximum(cnt, 1.0)
        return jnp.where(cnt > 0, mean, cent)

    cent = lax.fori_loop(0, _KM_ITERS, step, cent0)

    s = -dist(cent)
    smax = jnp.max(s, axis=1, keepdims=True)
    e = jnp.exp(s - smax)
    S = e / jnp.sum(e, axis=1, keepdims=True)

    xp = lax.dot_general(S, xb[:, :, _D:], (((2,), (1,)), ((0,), (0,))),
                         precision=lax.Precision.HIGHEST)
    xout_ref[0] = jnp.concatenate([cent, xp], axis=2)
    scol_ref[0] = S
    srow_ref[0] = jnp.transpose(S, (0, 2, 1))


def _pool_call(xg):
    ng, nc = xg.shape[0], xg.shape[3]
    return pl.pallas_call(
        _pool_body,
        grid=(ng,),
        in_specs=[
            pl.BlockSpec((1, _G, _N, nc), lambda b: (b, 0, 0, 0)),
        ],
        out_specs=[
            pl.BlockSpec((1, _G, _M, nc), lambda b: (b, 0, 0, 0)),
            pl.BlockSpec((1, _G, _N, _M), lambda b: (b, 0, 0, 0)),
            pl.BlockSpec((1, _G, _M, _N), lambda b: (b, 0, 0, 0)),
        ],
        out_shape=[
            jax.ShapeDtypeStruct((ng, _G, _M, nc), jnp.float32),
            jax.ShapeDtypeStruct((ng, _G, _N, _M), jnp.float32),
            jax.ShapeDtypeStruct((ng, _G, _M, _N), jnp.float32),
        ],
    )(xg)


def _scatter_call(s_rows, ep, zrows):
    mesh = plsc.VectorSubcoreMesh(core_axis_name="c", subcore_axis_name="s")

    @functools.partial(
        pl.kernel,
        out_type=jax.ShapeDtypeStruct((2 * _BN, _M), jnp.float32),
        mesh=mesh,
        scratch_types=[
            pltpu.VMEM((_NCHUNK, _CHUNK), jnp.int32),
            pltpu.VMEM((_NCHUNK, _CHUNK), jnp.int32),
            pltpu.VMEM((_NBUF, _CHUNK, _M), jnp.float32),
            pltpu.VMEM_SHARED((_ACC_ROWS, _M), jnp.float32),
            pltpu.SemaphoreType.DMA,
        ],
        compiler_params=pltpu.CompilerParams(use_tc_tiling_on_sc=False),
    )
    def scatter_kernel(s_hbm, e_hbm, z_hbm, out_hbm,
                       idx0_v, idx1_v, rows_v, acc_sh, sem):
        cid = lax.axis_index("c")
        sid = lax.axis_index("s")
        wid = sid * 2 + cid

        @pl.when(sid < 15)
        def _():
            pltpu.sync_copy(z_hbm.at[pl.ds(0, _STRIPE)],
                            acc_sh.at[pl.ds(sid * _STRIPE, _STRIPE)])

        @pl.when(sid == 15)
        def _():
            pltpu.sync_copy(z_hbm, acc_sh.at[pl.ds(15 * _STRIPE, _ZTAIL)])

        plsc.subcore_barrier()

        pltpu.sync_copy(e_hbm.at[0, wid], idx0_v)
        pltpu.sync_copy(e_hbm.at[1, wid], idx1_v)

        def gather(j, b):
            return pltpu.make_async_copy(
                s_hbm.at[idx1_v.at[j]], rows_v.at[b], sem)

        for b in range(_NBUF):
            gather(b, b).start()

        def group(g, carry):
            for b in range(_NBUF):
                j = g * _NBUF + b
                gather(j, b).wait()
                pltpu.sync_copy(rows_v.at[b], acc_sh.at[idx0_v.at[j]],
                                add=True)

                @pl.when(j + _NBUF < _NCHUNK)
                def _():
                    gather(j + _NBUF, b).start()
            return carry

        lax.fori_loop(0, _NCHUNK // _NBUF, group, 0)
        plsc.subcore_barrier()

        @pl.when(sid < 15)
        def _():
            pltpu.sync_copy(acc_sh.at[pl.ds(sid * _STRIPE, _STRIPE)],
                            out_hbm.at[pl.ds(cid * _BN + sid * _STRIPE, _STRIPE)])

        @pl.when(sid == 15)
        def _():
            pltpu.sync_copy(acc_sh.at[pl.ds(15 * _STRIPE, _CTAIL)],
                            out_hbm.at[pl.ds(cid * _BN + 15 * _STRIPE, _CTAIL)])

    return scatter_kernel(s_rows, ep, zrows)


def _amm_body(scol_ref, as4_ref, dst_ref):
    S = scol_ref[0]
    asb = as4_ref[0, 0] + as4_ref[1, 0]
    a = lax.dot_general(S, asb, (((1,), (0,)), ((), ())),
                        precision=lax.Precision.HIGHEST)
    liota = lax.broadcasted_iota(jnp.int32, (_M, _M), 1)
    cols = []
    for _ in range(_TOPK):
        vmax = jnp.max(a, axis=1, keepdims=True)
        first = jnp.min(jnp.where(a == vmax, liota, _M), axis=1, keepdims=True)
        cols.append(first)
        a = jnp.where(liota == first, -jnp.inf, a)
    dst_ref[0] = jnp.concatenate(cols, axis=1)


def _amm_call(s_cols, as4):
    return pl.pallas_call(
        _amm_body,
        grid=(_B,),
        in_specs=[
            pl.BlockSpec((1, _M, _N), lambda b: (b, 0, 0)),
            pl.BlockSpec((2, 1, _N, _M), lambda b: (0, b, 0, 0)),
        ],
        out_specs=pl.BlockSpec((1, _M, _TOPK), lambda b: (b, 0, 0)),
        out_shape=jax.ShapeDtypeStruct((_B, _M, _TOPK), jnp.int32),
    )(s_cols, as4)


def kernel(x, e_, b_):
    nc = x.shape[1]
    xg = x.reshape(_B // _G, _G, _N, nc)

    xout, s_rows, s_cols = _pool_call(xg)

    pad = _EPAD - _E
    padcols = jnp.concatenate(
        [jnp.full((1, pad), _DUMMY_ROW, jnp.int32),
         jnp.zeros((1, pad), jnp.int32)], axis=0)
    ep = jnp.concatenate([e_, padcols], axis=1).reshape(2, _NW, _NCHUNK, _CHUNK)
    zrows = jnp.zeros((_ZTAIL, _M), jnp.float32)
    as2 = _scatter_call(s_rows.reshape(_BN, _M), ep, zrows)

    dst = _amm_call(s_cols.reshape(_B, _M, _N), as2.reshape(2, _B, _N, _M))

    x_out = xout.reshape(_B * _M, nc)
    off = (jnp.arange(_B, dtype=jnp.int32) * _M).reshape(_B, 1, 1)
    src = jnp.broadcast_to(
        jnp.arange(_M, dtype=jnp.int32)[None, :, None], (_B, _M, _TOPK)) + off
    e_out = jnp.stack([src.reshape(-1), (dst + off).reshape(-1)], axis=0)
    b_out = jnp.repeat(jnp.arange(_B, dtype=jnp.int32), _M)
    return (x_out, e_out, b_out)


# speedup vs baseline: 1.4897x; 1.0010x over previous
import functools

import jax
import jax.numpy as jnp
from jax import lax
from jax.experimental import pallas as pl
from jax.experimental.pallas import tpu as pltpu
from jax.experimental.pallas import tpu_sc as plsc

_M = 16
_B = 8
_G = 8
_N = 1250
_BN = _B * _N
_D = 3
_E = 160000
_KM_ITERS = 20
_TOPK = 4

_NW = 32
_CHUNK = 128
_NCHUNK = 40
_EPW = _CHUNK * _NCHUNK
_EPAD = _NW * _EPW
_NBUF = 8
_AHEAD = 4
_DUMMY_ROW = _BN
_ACC_ROWS = _BN + _M
_STRIPE = 624
_ZTAIL = _ACC_ROWS - 15 * _STRIPE
_CTAIL = _BN - 15 * _STRIPE


def _pool_body(x_ref, xout_ref, srow_ref, scol_ref):
    xb = x_ref[0]
    x3t = jnp.transpose(xb[:, :, :_D], (0, 2, 1))
    cent0 = xb[:, :_M, :_D]

    def dist(cent):
        d = None
        for c in range(_D):
            diff = x3t[:, c:c + 1, :] - cent[:, :, c:c + 1]
            sq = diff * diff
            d = sq if d is None else d + sq
        return d

    miota = lax.broadcasted_iota(jnp.int32, (_G, _M, _N), 1)

    def step(_, cent):
        d = dist(cent)
        dmin = jnp.min(d, axis=1, keepdims=True)
        first = jnp.min(jnp.where(d == dmin, miota, _M), axis=1, keepdims=True)
        p = (miota == first).astype(jnp.float32)
        sums = lax.dot_general(p, x3t, (((2,), (2,)), ((0,), (0,))),
                               precision=lax.Precision.HIGHEST)
        cnt = jnp.sum(p, axis=2, keepdims=True)
        mean = sums / jnp.maximum(cnt, 1.0)
        return jnp.where(cnt > 0, mean, cent)

    cent = lax.fori_loop(0, _KM_ITERS, step, cent0)

    s = -dist(cent)
    smax = jnp.max(s, axis=1, keepdims=True)
    e = jnp.exp(s - smax)
    S = e / jnp.sum(e, axis=1, keepdims=True)

    xp = lax.dot_general(S, xb[:, :, _D:], (((2,), (1,)), ((0,), (0,))),
                         precision=lax.Precision.HIGHEST)
    xout_ref[0] = jnp.concatenate([cent, xp], axis=2)
    scol_ref[0] = S
    srow_ref[0] = jnp.transpose(S, (0, 2, 1))


def _pool_call(xg):
    ng, nc = xg.shape[0], xg.shape[3]
    return pl.pallas_call(
        _pool_body,
        grid=(ng,),
        in_specs=[
            pl.BlockSpec((1, _G, _N, nc), lambda b: (b, 0, 0, 0)),
        ],
        out_specs=[
            pl.BlockSpec((1, _G, _M, nc), lambda b: (b, 0, 0, 0)),
            pl.BlockSpec((1, _G, _N, _M), lambda b: (b, 0, 0, 0)),
            pl.BlockSpec((1, _G, _M, _N), lambda b: (b, 0, 0, 0)),
        ],
        out_shape=[
            jax.ShapeDtypeStruct((ng, _G, _M, nc), jnp.float32),
            jax.ShapeDtypeStruct((ng, _G, _N, _M), jnp.float32),
            jax.ShapeDtypeStruct((ng, _G, _M, _N), jnp.float32),
        ],
    )(xg)


def _scatter_call(s_rows, ep, zrows):
    mesh = plsc.VectorSubcoreMesh(core_axis_name="c", subcore_axis_name="s")

    @functools.partial(
        pl.kernel,
        out_type=jax.ShapeDtypeStruct((2 * _BN, _M), jnp.float32),
        mesh=mesh,
        scratch_types=[
            pltpu.VMEM((_NCHUNK, _CHUNK), jnp.int32),
            pltpu.VMEM((_NCHUNK, _CHUNK), jnp.int32),
            pltpu.VMEM((_NBUF, _CHUNK, _M), jnp.float32),
            pltpu.VMEM_SHARED((_ACC_ROWS, _M), jnp.float32),
            pltpu.SemaphoreType.DMA,
            pltpu.SemaphoreType.DMA,
        ],
        compiler_params=pltpu.CompilerParams(use_tc_tiling_on_sc=False),
    )
    def scatter_kernel(s_hbm, e_hbm, z_hbm, out_hbm,
                       idx0_v, idx1_v, rows_v, acc_sh, gsem, ssem):
        cid = lax.axis_index("c")
        sid = lax.axis_index("s")
        wid = sid * 2 + cid

        @pl.when(sid < 15)
        def _():
            pltpu.sync_copy(z_hbm.at[pl.ds(0, _STRIPE)],
                            acc_sh.at[pl.ds(sid * _STRIPE, _STRIPE)])

        @pl.when(sid == 15)
        def _():
            pltpu.sync_copy(z_hbm, acc_sh.at[pl.ds(15 * _STRIPE, _ZTAIL)])

        plsc.subcore_barrier()

        pltpu.sync_copy(e_hbm.at[0, wid], idx0_v)
        pltpu.sync_copy(e_hbm.at[1, wid], idx1_v)

        def gather(j, b):
            return pltpu.make_async_copy(
                s_hbm.at[idx1_v.at[j]], rows_v.at[b], gsem)

        def scat_start(j, b):
            pltpu.async_copy(rows_v.at[b], acc_sh.at[idx0_v.at[j]], ssem,
                             add=True)

        def scat_wait(j, b):
            pltpu.make_async_copy(rows_v.at[b], acc_sh.at[idx0_v.at[j]],
                                  ssem).wait()

        for b in range(_AHEAD):
            gather(b, b).start()

        def group(g, carry):
            for b in range(_NBUF):
                j = g * _NBUF + b
                gather(j, b).wait()
                scat_start(j, b)

                @pl.when(j >= _AHEAD)
                def _():
                    scat_wait(j - _AHEAD, (b - _AHEAD) % _NBUF)

                @pl.when(j + _AHEAD < _NCHUNK)
                def _():
                    gather(j + _AHEAD, (b + _AHEAD) % _NBUF).start()
            return carry

        lax.fori_loop(0, _NCHUNK // _NBUF, group, 0)
        for k in range(_AHEAD):
            j = _NCHUNK - _AHEAD + k
            scat_wait(j, j % _NBUF)
        plsc.subcore_barrier()

        @pl.when(sid < 15)
        def _():
            pltpu.sync_copy(acc_sh.at[pl.ds(sid * _STRIPE, _STRIPE)],
                            out_hbm.at[pl.ds(cid * _BN + sid * _STRIPE, _STRIPE)])

        @pl.when(sid == 15)
        def _():
            pltpu.sync_copy(acc_sh.at[pl.ds(15 * _STRIPE, _CTAIL)],
                            out_hbm.at[pl.ds(cid * _BN + 15 * _STRIPE, _CTAIL)])

    return scatter_kernel(s_rows, ep, zrows)


def _amm_body(scol_ref, as4_ref, dst_ref):
    S = scol_ref[0]
    asb = as4_ref[0, 0] + as4_ref[1, 0]
    a = lax.dot_general(S, asb, (((1,), (0,)), ((), ())),
                        precision=lax.Precision.HIGHEST)
    liota = lax.broadcasted_iota(jnp.int32, (_M, _M), 1)
    cols = []
    for _ in range(_TOPK):
        vmax = jnp.max(a, axis=1, keepdims=True)
        first = jnp.min(jnp.where(a == vmax, liota, _M), axis=1, keepdims=True)
        cols.append(first)
        a = jnp.where(liota == first, -jnp.inf, a)
    dst_ref[0] = jnp.concatenate(cols, axis=1)


def _amm_call(s_cols, as4):
    return pl.pallas_call(
        _amm_body,
        grid=(_B,),
        in_specs=[
            pl.BlockSpec((1, _M, _N), lambda b: (b, 0, 0)),
            pl.BlockSpec((2, 1, _N, _M), lambda b: (0, b, 0, 0)),
        ],
        out_specs=pl.BlockSpec((1, _M, _TOPK), lambda b: (b, 0, 0)),
        out_shape=jax.ShapeDtypeStruct((_B, _M, _TOPK), jnp.int32),
    )(s_cols, as4)


def kernel(x, e_, b_):
    nc = x.shape[1]
    xg = x.reshape(_B // _G, _G, _N, nc)

    xout, s_rows, s_cols = _pool_call(xg)

    pad = _EPAD - _E
    padcols = jnp.concatenate(
        [jnp.full((1, pad), _DUMMY_ROW, jnp.int32),
         jnp.zeros((1, pad), jnp.int32)], axis=0)
    ep = jnp.concatenate([e_, padcols], axis=1).reshape(2, _NW, _NCHUNK, _CHUNK)
    zrows = jnp.zeros((_ZTAIL, _M), jnp.float32)
    as2 = _scatter_call(s_rows.reshape(_BN, _M), ep, zrows)

    dst = _amm_call(s_cols.reshape(_B, _M, _N), as2.reshape(2, _B, _N, _M))

    x_out = xout.reshape(_B * _M, nc)
    off = (jnp.arange(_B, dtype=jnp.int32) * _M).reshape(_B, 1, 1)
    src = jnp.broadcast_to(
        jnp.arange(_M, dtype=jnp.int32)[None, :, None], (_B, _M, _TOPK)) + off
    e_out = jnp.stack([src.reshape(-1), (dst + off).reshape(-1)], axis=0)
    b_out = jnp.repeat(jnp.arange(_B, dtype=jnp.int32), _M)
    return (x_out, e_out, b_out)
